# Initial kernel scaffold; baseline (speedup 1.0000x reference)
import jax
import jax.numpy as jnp
from jax import lax
from jax.experimental import pallas as pl
from jax.experimental.pallas import tpu as pltpu
from jax.experimental.pallas import tpu_sc as plsc

N = 10000
E = 160000
D = 128
NS = 16
LANES = 16
EPT = E // NS
B = 80
NCH = EPT // B
GPC = B // LANES
ROWS_PT = N // NS
DEN_PAD = 10240
BN = 1000
NBLK = N // BN


def _tc_proj_body(x_ref, w_ref, al_ref, ar_ref,
                  feat_ref, el_ref, er_ref, c_ref, sm):
    i = pl.program_id(1)
    feat = lax.dot_general(x_ref[...], w_ref[0], (((1,), (1,)), ((), ())),
                           preferred_element_type=jnp.float32)
    feat_ref[...] = feat
    el = jnp.sum(feat * al_ref[0], axis=1, keepdims=True)
    er = jnp.sum(feat * ar_ref[0], axis=1, keepdims=True)
    el_ref[...] = el
    er_ref[...] = er
    bl = jnp.max(el)
    br = jnp.max(er)

    @pl.when(i == 0)
    def _():
        sm[0] = bl
        sm[1] = br

    @pl.when(i > 0)
    def _():
        sm[0] = jnp.maximum(sm[0], bl)
        sm[1] = jnp.maximum(sm[1], br)

    @pl.when(i == NBLK - 1)
    def _():
        c_ref[...] = jnp.full((1, D), jnp.maximum(sm[0] + sm[1], 0.0),
                              jnp.float32)


_tc_proj = pl.pallas_call(
    _tc_proj_body,
    grid=(2, NBLK),
    in_specs=[
        pl.BlockSpec((BN, D), lambda e, i: (i, 0)),
        pl.BlockSpec((1, D, D), lambda e, i: (e, 0, 0)),
        pl.BlockSpec((1, 1, D), lambda e, i: (e, 0, 0)),
        pl.BlockSpec((1, 1, D), lambda e, i: (e, 0, 0)),
    ],
    out_specs=[
        pl.BlockSpec((BN, D), lambda e, i: (e * NBLK + i, 0)),
        pl.BlockSpec((BN, 1), lambda e, i: (e * NBLK + i, 0)),
        pl.BlockSpec((BN, 1), lambda e, i: (e * NBLK + i, 0)),
        pl.BlockSpec((1, D), lambda e, i: (e, 0)),
    ],
    out_shape=[
        jax.ShapeDtypeStruct((2 * N, D), jnp.float32),
        jax.ShapeDtypeStruct((2 * N, 1), jnp.float32),
        jax.ShapeDtypeStruct((2 * N, 1), jnp.float32),
        jax.ShapeDtypeStruct((2, D), jnp.float32),
    ],
    scratch_shapes=[pltpu.SMEM((2,), jnp.float32)],
)


def _sc_body(u_hbm, v_hbm, el_hbm, er_hbm, c_hbm, feat_hbm, h_hbm,
             u_v, v_v, ug_v, p_v, el_v, er_v, den_v, rows_v, zer_v, c_v,
             acc_sh, den_sh, sem):
    cid = lax.axis_index("c")
    sid = lax.axis_index("s")
    zero16 = jnp.zeros((LANES,), jnp.float32)

    def _zrow(r, _):
        for k in range(D // LANES):
            rows_v[r, pl.ds(k * LANES, LANES)] = zero16
        return 0
    lax.fori_loop(0, B, _zrow, 0)

    def _zden(k, _):
        zer_v[pl.ds(k * LANES, LANES)] = zero16
        return 0
    lax.fori_loop(0, 640 // LANES, _zden, 0)

    pltpu.sync_copy(zer_v, den_sh.at[pl.ds(sid * 640, 640)])
    row0 = sid * ROWS_PT
    for t in range(7):
        pltpu.sync_copy(rows_v, acc_sh.at[pl.ds(row0 + t * B, B)])
    pltpu.sync_copy(rows_v.at[pl.ds(0, 65)],
                    acc_sh.at[pl.ds(row0 + 7 * B, 65)])

    ebase = cid * E + sid * EPT

    def _stage(j, _):
        off = pl.multiple_of(ebase + j * B, 8)
        pltpu.sync_copy(u_hbm.at[pl.ds(off, B)], u_v.at[j])
        pltpu.sync_copy(v_hbm.at[pl.ds(off, B)], v_v.at[j])
        return 0
    lax.fori_loop(0, NCH, _stage, 0)

    nbase = pl.multiple_of(cid * N, 8)
    pltpu.sync_copy(el_hbm.at[pl.ds(nbase, N)], el_v)
    pltpu.sync_copy(er_hbm.at[pl.ds(nbase, N)], er_v)
    pltpu.sync_copy(c_hbm.at[pl.ds(pl.multiple_of(cid * D, 8), LANES)], c_v)
    cvec = c_v[...]
    plsc.subcore_barrier()

    cofs = cid * N

    def _p_chunk(j, _):
        def _grp(g, _2):
            o = pl.multiple_of(g * LANES, 8)
            u16 = u_v[j, pl.ds(o, LANES)]
            v16 = v_v[j, pl.ds(o, LANES)]
            elu = plsc.load_gather(el_v, [u16])
            erv = plsc.load_gather(er_v, [v16])
            s = elu + erv
            s = jnp.where(s >= 0.0, s, 0.2 * s)
            p_v[j, pl.ds(o, LANES)] = jnp.exp(s - cvec)
            ug_v[j, pl.ds(o, LANES)] = u16 + cofs
            return 0
        lax.fori_loop(0, GPC, _grp, 0)
        pltpu.sync_copy(p_v.at[j], den_sh.at[v_v.at[j]], add=True)
        return 0
    lax.fori_loop(0, NCH, _p_chunk, 0)
    plsc.subcore_barrier()

    pltpu.sync_copy(den_sh.at[pl.ds(0, N)], den_v)

    def _alpha_chunk(j, _):
        def _grp(g, _2):
            o = pl.multiple_of(g * LANES, 8)
            v16 = v_v[j, pl.ds(o, LANES)]
            dv = plsc.load_gather(den_v, [v16])
            p_v[j, pl.ds(o, LANES)] = p_v[j, pl.ds(o, LANES)] / dv
            return 0
        lax.fori_loop(0, GPC, _grp, 0)
        return 0
    lax.fori_loop(0, NCH, _alpha_chunk, 0)

    def _agg_chunk(j, _):
        pltpu.async_copy(feat_hbm.at[ug_v.at[j]], rows_v, sem).wait()

        def _scale(r, _2):
            jj = jnp.full((LANES,), j, jnp.int32)
            rr = jnp.full((LANES,), r, jnp.int32)
            af = plsc.load_gather(p_v, [jj, rr])
            for k in range(D // LANES):
                sl = pl.ds(k * LANES, LANES)
                rows_v[r, sl] = rows_v[r, sl] * af
            return 0
        lax.fori_loop(0, B, _scale, 0)
        pltpu.sync_copy(rows_v, acc_sh.at[v_v.at[j]], add=True)
        return 0
    lax.fori_loop(0, NCH, _agg_chunk, 0)
    plsc.subcore_barrier()

    hb = cid * N + sid * ROWS_PT
    pltpu.sync_copy(acc_sh.at[pl.ds(sid * ROWS_PT, ROWS_PT)],
                    h_hbm.at[pl.ds(hb, ROWS_PT)])


_sc_agg = pl.kernel(
    _sc_body,
    out_type=jax.ShapeDtypeStruct((2 * N, D), jnp.float32),
    mesh=plsc.VectorSubcoreMesh(core_axis_name="c", subcore_axis_name="s"),
    scratch_types=[
        pltpu.VMEM((NCH, B), jnp.int32),
        pltpu.VMEM((NCH, B), jnp.int32),
        pltpu.VMEM((NCH, B), jnp.int32),
        pltpu.VMEM((NCH, B), jnp.float32),
        pltpu.VMEM((N,), jnp.float32),
        pltpu.VMEM((N,), jnp.float32),
        pltpu.VMEM((N,), jnp.float32),
        pltpu.VMEM((B, D), jnp.float32),
        pltpu.VMEM((640,), jnp.float32),
        pltpu.VMEM((LANES,), jnp.float32),
        pltpu.VMEM_SHARED((N, D), jnp.float32),
        pltpu.VMEM_SHARED((DEN_PAD,), jnp.float32),
        pltpu.SemaphoreType.DMA,
    ],
)


def _tc_wl_body(x_ref, h0_ref, h1_ref, wx_ref, wd_ref, b_ref, o_ref):
    cdims = (((1,), (1,)), ((), ()))
    t = lax.dot_general(x_ref[...], wx_ref[...], cdims,
                        preferred_element_type=jnp.float32)
    bias = b_ref[...]
    a1 = h0_ref[...] + bias
    d1 = jnp.maximum(t + lax.dot_general(a1, wd_ref[...], cdims,
                                         preferred_element_type=jnp.float32),
                     0.0)
    a2 = d1 + h1_ref[...] + bias
    o_ref[...] = jnp.maximum(
        t + lax.dot_general(a2, wd_ref[...], cdims,
                            preferred_element_type=jnp.float32), 0.0)


_tc_wl = pl.pallas_call(
    _tc_wl_body,
    grid=(NBLK,),
    in_specs=[
        pl.BlockSpec((BN, D), lambda i: (i, 0)),
        pl.BlockSpec((BN, D), lambda i: (i, 0)),
        pl.BlockSpec((BN, D), lambda i: (NBLK + i, 0)),
        pl.BlockSpec((D, D), lambda i: (0, 0)),
        pl.BlockSpec((D, D), lambda i: (0, 0)),
        pl.BlockSpec((1, D), lambda i: (0, 0)),
    ],
    out_specs=pl.BlockSpec((BN, D), lambda i: (i, 0)),
    out_shape=jax.ShapeDtypeStruct((N, D), jnp.float32),
)


def kernel(x, edge_index0, edge_index1, W0, attn_l0, attn_r0,
           W1, attn_l1, attn_r1, wl_W, bias):
    Wst = jnp.stack([W0, W1])
    ALst = jnp.stack([attn_l0, attn_l1]).reshape(2, 1, D)
    ARst = jnp.stack([attn_r0, attn_r1]).reshape(2, 1, D)
    FEAT, EL, ER, CC = _tc_proj(x, Wst, ALst, ARst)
    U = jnp.concatenate([edge_index0[0], edge_index1[0]])
    V = jnp.concatenate([edge_index0[1], edge_index1[1]])
    H = _sc_agg(U, V, EL.reshape(2 * N), ER.reshape(2 * N),
                CC.reshape(2 * D), FEAT)
    wlx = wl_W[:, :D]
    wld = wl_W[:, D:]
    return _tc_wl(x, H, H, wlx, wld, bias.reshape(1, D))


# speedup vs baseline: 12.7847x; 12.7847x over previous
import jax
import jax.numpy as jnp
from jax import lax
from jax.experimental import pallas as pl
from jax.experimental.pallas import tpu as pltpu
from jax.experimental.pallas import tpu_sc as plsc

N = 10000
E = 160000
D = 128
NS = 16
LANES = 16
EPT = E // NS
B = 80
NCH = EPT // B
GPC = B // LANES
ACC_PAD = 10240
ROWS_PT = ACC_PAD // NS
DEN_PAD = 10240
BN = 1000
NBLK = N // BN


def _tc_proj_body(x_ref, w_ref, al_ref, ar_ref,
                  feat_ref, el_ref, er_ref, c_ref, sm):
    i = pl.program_id(1)
    feat = lax.dot_general(x_ref[...], w_ref[0], (((1,), (1,)), ((), ())),
                           preferred_element_type=jnp.float32)
    feat_ref[...] = feat
    el = jnp.sum(feat * al_ref[0], axis=1, keepdims=True)
    er = jnp.sum(feat * ar_ref[0], axis=1, keepdims=True)
    el_ref[...] = el
    er_ref[...] = er
    bl = jnp.max(el)
    br = jnp.max(er)

    @pl.when(i == 0)
    def _():
        sm[0] = bl
        sm[1] = br

    @pl.when(i > 0)
    def _():
        sm[0] = jnp.maximum(sm[0], bl)
        sm[1] = jnp.maximum(sm[1], br)

    @pl.when(i == NBLK - 1)
    def _():
        c_ref[...] = jnp.full((8, D), jnp.maximum(sm[0] + sm[1], 0.0),
                              jnp.float32)


_tc_proj = pl.pallas_call(
    _tc_proj_body,
    grid=(2, NBLK),
    in_specs=[
        pl.BlockSpec((BN, D), lambda e, i: (i, 0)),
        pl.BlockSpec((1, D, D), lambda e, i: (e, 0, 0)),
        pl.BlockSpec((1, 1, D), lambda e, i: (e, 0, 0)),
        pl.BlockSpec((1, 1, D), lambda e, i: (e, 0, 0)),
    ],
    out_specs=[
        pl.BlockSpec((BN, D), lambda e, i: (e * NBLK + i, 0)),
        pl.BlockSpec((BN, 1), lambda e, i: (e * NBLK + i, 0)),
        pl.BlockSpec((BN, 1), lambda e, i: (e * NBLK + i, 0)),
        pl.BlockSpec((8, D), lambda e, i: (e, 0)),
    ],
    out_shape=[
        jax.ShapeDtypeStruct((2 * N, D), jnp.float32),
        jax.ShapeDtypeStruct((2 * N, 1), jnp.float32),
        jax.ShapeDtypeStruct((2 * N, 1), jnp.float32),
        jax.ShapeDtypeStruct((16, D), jnp.float32),
    ],
    scratch_shapes=[pltpu.SMEM((2,), jnp.float32)],
)


def _sc_body(u_hbm, v_hbm, el_hbm, er_hbm, c_hbm, feat_hbm,
             h_hbm, den_hbm,
             u_c, v_c, ug_c, vg_c, p_v, elu_c, erv_c, rows_v, c_v,
             acc_sh, den_sh, sem):
    cid = lax.axis_index("c")
    sid = lax.axis_index("s")
    zero16 = jnp.zeros((LANES,), jnp.float32)

    def _zrow(r, _):
        for k in range(D // LANES):
            rows_v[r, pl.ds(k * LANES, LANES)] = zero16
        return 0
    lax.fori_loop(0, B, _zrow, 0)

    row0 = pl.multiple_of(sid * ROWS_PT, 8)
    for t in range(ROWS_PT // B):
        pltpu.sync_copy(rows_v, acc_sh.at[pl.ds(row0 + t * B, B)])
    for t in range(5):
        pltpu.sync_copy(rows_v.at[0],
                        den_sh.at[pl.ds(sid * 640 + t * D, D)])

    pltpu.sync_copy(c_hbm.at[pl.ds(pl.multiple_of(cid * 8 * D, 8), LANES)],
                    c_v)
    cvec = c_v[...]
    plsc.subcore_barrier()

    ebase = cid * E + sid * EPT
    cofs = cid * N

    def _p_chunk(j, _):
        off = pl.multiple_of(ebase + j * B, 8)
        pltpu.sync_copy(u_hbm.at[pl.ds(off, B)], u_c)
        pltpu.sync_copy(v_hbm.at[pl.ds(off, B)], v_c)

        def _idx(g, _2):
            o = pl.multiple_of(g * LANES, 8)
            ug_c[pl.ds(o, LANES)] = u_c[pl.ds(o, LANES)] + cofs
            vg_c[pl.ds(o, LANES)] = v_c[pl.ds(o, LANES)] + cofs
            return 0
        lax.fori_loop(0, GPC, _idx, 0)
        pltpu.async_copy(el_hbm.at[ug_c], elu_c, sem).wait()
        pltpu.async_copy(er_hbm.at[vg_c], erv_c, sem).wait()

        def _grp(g, _2):
            o = pl.multiple_of(g * LANES, 8)
            s = elu_c[pl.ds(o, LANES)] + erv_c[pl.ds(o, LANES)]
            s = jnp.where(s >= 0.0, s, 0.2 * s)
            p_v[j, pl.ds(o, LANES)] = jnp.exp(s - cvec)
            return 0
        lax.fori_loop(0, GPC, _grp, 0)
        pltpu.sync_copy(p_v.at[j], den_sh.at[v_c], add=True)
        return 0
    lax.fori_loop(0, NCH, _p_chunk, 0)
    plsc.subcore_barrier()

    def _agg_chunk(j, _):
        off = pl.multiple_of(ebase + j * B, 8)
        pltpu.sync_copy(u_hbm.at[pl.ds(off, B)], u_c)
        pltpu.sync_copy(v_hbm.at[pl.ds(off, B)], v_c)

        def _idx(g, _2):
            o = pl.multiple_of(g * LANES, 8)
            ug_c[pl.ds(o, LANES)] = u_c[pl.ds(o, LANES)] + cofs
            return 0
        lax.fori_loop(0, GPC, _idx, 0)
        pltpu.async_copy(feat_hbm.at[ug_c], rows_v, sem).wait()

        def _scale(r, _2):
            jj = jnp.full((LANES,), j, jnp.int32)
            rr = jnp.full((LANES,), r, jnp.int32)
            af = plsc.load_gather(p_v, [jj, rr])
            for k in range(D // LANES):
                sl = pl.ds(k * LANES, LANES)
                rows_v[r, sl] = rows_v[r, sl] * af
            return 0
        lax.fori_loop(0, B, _scale, 0)
        pltpu.sync_copy(rows_v, acc_sh.at[v_c], add=True)
        return 0
    lax.fori_loop(0, NCH, _agg_chunk, 0)
    plsc.subcore_barrier()

    pltpu.sync_copy(den_sh.at[pl.ds(sid * 640, 640)],
                    den_hbm.at[pl.ds(cid * DEN_PAD + sid * 640, 640)])

    @pl.when(sid < NS - 1)
    def _():
        hb = pl.multiple_of(cid * N + sid * ROWS_PT, 8)
        pltpu.sync_copy(acc_sh.at[pl.ds(row0, ROWS_PT)],
                        h_hbm.at[pl.ds(hb, ROWS_PT)])

    @pl.when(sid == NS - 1)
    def _():
        nrem = N - (NS - 1) * ROWS_PT
        hb = pl.multiple_of(cid * N + (NS - 1) * ROWS_PT, 8)
        pltpu.sync_copy(acc_sh.at[pl.ds(row0, nrem)],
                        h_hbm.at[pl.ds(hb, nrem)])


_sc_agg_built = None


def _sc_agg(*args):
    global _sc_agg_built
    if _sc_agg_built is None:
        _sc_agg_built = _build_sc_agg()
    return _sc_agg_built(*args)


def _build_sc_agg():
    return pl.kernel(
        _sc_body,
        out_type=(jax.ShapeDtypeStruct((2 * N, D), jnp.float32),
                  jax.ShapeDtypeStruct((2 * DEN_PAD,), jnp.float32)),
        mesh=plsc.VectorSubcoreMesh(core_axis_name="c", subcore_axis_name="s",
                                    num_cores=2, num_subcores=NS),
        compiler_params=pltpu.CompilerParams(needs_layout_passes=False),
        scratch_types=[
            pltpu.VMEM((B,), jnp.int32),
            pltpu.VMEM((B,), jnp.int32),
            pltpu.VMEM((B,), jnp.int32),
            pltpu.VMEM((B,), jnp.int32),
            pltpu.VMEM((NCH, B), jnp.float32),
            pltpu.VMEM((B,), jnp.float32),
            pltpu.VMEM((B,), jnp.float32),
            pltpu.VMEM((B, D), jnp.float32),
            pltpu.VMEM((LANES,), jnp.float32),
            pltpu.VMEM_SHARED((ACC_PAD, D), jnp.float32),
            pltpu.VMEM_SHARED((DEN_PAD,), jnp.float32),
            pltpu.SemaphoreType.DMA,
        ],
    )


def _tc_wl_body(x_ref, h0_ref, h1_ref, d0_ref, d1d_ref, wx_ref, wd_ref,
                b_ref, o_ref):
    cdims = (((1,), (1,)), ((), ()))
    den0 = d0_ref[...]
    den1 = d1d_ref[...]
    h0 = h0_ref[...] / jnp.where(den0 == 0.0, 1.0, den0)
    h1 = h1_ref[...] / jnp.where(den1 == 0.0, 1.0, den1)
    t = lax.dot_general(x_ref[...], wx_ref[...], cdims,
                        preferred_element_type=jnp.float32)
    bias = b_ref[...]
    a1 = h0 + bias
    d1 = jnp.maximum(t + lax.dot_general(a1, wd_ref[...], cdims,
                                         preferred_element_type=jnp.float32),
                     0.0)
    a2 = d1 + h1 + bias
    o_ref[...] = jnp.maximum(
        t + lax.dot_general(a2, wd_ref[...], cdims,
                            preferred_element_type=jnp.float32), 0.0)


_tc_wl = pl.pallas_call(
    _tc_wl_body,
    grid=(NBLK,),
    in_specs=[
        pl.BlockSpec((BN, D), lambda i: (i, 0)),
        pl.BlockSpec((BN, D), lambda i: (i, 0)),
        pl.BlockSpec((BN, D), lambda i: (NBLK + i, 0)),
        pl.BlockSpec((BN, 1), lambda i: (i, 0)),
        pl.BlockSpec((BN, 1), lambda i: (i, 0)),
        pl.BlockSpec((D, D), lambda i: (0, 0)),
        pl.BlockSpec((D, D), lambda i: (0, 0)),
        pl.BlockSpec((1, D), lambda i: (0, 0)),
    ],
    out_specs=pl.BlockSpec((BN, D), lambda i: (i, 0)),
    out_shape=jax.ShapeDtypeStruct((N, D), jnp.float32),
)


def kernel(x, edge_index0, edge_index1, W0, attn_l0, attn_r0,
           W1, attn_l1, attn_r1, wl_W, bias):
    Wst = jnp.stack([W0, W1])
    ALst = jnp.stack([attn_l0, attn_l1]).reshape(2, 1, D)
    ARst = jnp.stack([attn_r0, attn_r1]).reshape(2, 1, D)
    FEAT, EL, ER, CC = _tc_proj(x, Wst, ALst, ARst)
    U = jnp.concatenate([edge_index0[0], edge_index1[0]])
    V = jnp.concatenate([edge_index0[1], edge_index1[1]])
    H, DEN = _sc_agg(U, V, EL.reshape(2 * N), ER.reshape(2 * N),
                     CC.reshape(16 * D), FEAT)
    den0 = DEN[:N].reshape(N, 1)
    den1 = DEN[DEN_PAD:DEN_PAD + N].reshape(N, 1)
    wlx = wl_W[:, :D]
    wld = wl_W[:, D:]
    return _tc_wl(x, H, H, den0, den1, wlx, wld, bias.reshape(1, D))


# speedup vs baseline: 15.9083x; 1.2443x over previous
import jax
import jax.numpy as jnp
from jax import lax
from jax.experimental import pallas as pl
from jax.experimental.pallas import tpu as pltpu
from jax.experimental.pallas import tpu_sc as plsc

N = 10000
E = 160000
D = 128
NS = 16
LANES = 16
EPT = E // NS
B = 80
NCH = EPT // B
GPC = B // LANES
ACC_PAD = 10240
ROWS_PT = ACC_PAD // NS
DEN_PAD = 10240
BN = 1000
NBLK = N // BN


def _tc_proj_body(x_ref, w_ref, al_ref, ar_ref,
                  feat_ref, el_ref, er_ref, c_ref, sm):
    i = pl.program_id(1)
    feat = lax.dot_general(x_ref[...], w_ref[0], (((1,), (1,)), ((), ())),
                           preferred_element_type=jnp.float32)
    feat_ref[...] = feat
    el = jnp.sum(feat * al_ref[0], axis=1, keepdims=True)
    er = jnp.sum(feat * ar_ref[0], axis=1, keepdims=True)
    el_ref[...] = el
    er_ref[...] = er
    bl = jnp.max(el)
    br = jnp.max(er)

    @pl.when(i == 0)
    def _():
        sm[0] = bl
        sm[1] = br

    @pl.when(i > 0)
    def _():
        sm[0] = jnp.maximum(sm[0], bl)
        sm[1] = jnp.maximum(sm[1], br)

    @pl.when(i == NBLK - 1)
    def _():
        c_ref[...] = jnp.full((8, D), jnp.maximum(sm[0] + sm[1], 0.0),
                              jnp.float32)


_tc_proj = pl.pallas_call(
    _tc_proj_body,
    grid=(2, NBLK),
    in_specs=[
        pl.BlockSpec((BN, D), lambda e, i: (i, 0)),
        pl.BlockSpec((1, D, D), lambda e, i: (e, 0, 0)),
        pl.BlockSpec((1, 1, D), lambda e, i: (e, 0, 0)),
        pl.BlockSpec((1, 1, D), lambda e, i: (e, 0, 0)),
    ],
    out_specs=[
        pl.BlockSpec((BN, D), lambda e, i: (e * NBLK + i, 0)),
        pl.BlockSpec((BN, 1), lambda e, i: (e * NBLK + i, 0)),
        pl.BlockSpec((BN, 1), lambda e, i: (e * NBLK + i, 0)),
        pl.BlockSpec((8, D), lambda e, i: (e, 0)),
    ],
    out_shape=[
        jax.ShapeDtypeStruct((2 * N, D), jnp.float32),
        jax.ShapeDtypeStruct((2 * N, 1), jnp.float32),
        jax.ShapeDtypeStruct((2 * N, 1), jnp.float32),
        jax.ShapeDtypeStruct((16, D), jnp.float32),
    ],
    scratch_shapes=[pltpu.SMEM((2,), jnp.float32)],
)


def _sc_body(u_hbm, v_hbm, el_hbm, er_hbm, c_hbm, feat_hbm,
             h_hbm, den_hbm, p_hbm,
             u_c, v2_c, ug2_c, vg2_c, p2_c, elu2_c, erv2_c, rows_v, c_v,
             acc_sh, den_sh, sem):
    cid = lax.axis_index("c")
    sid = lax.axis_index("s")
    zero16 = jnp.zeros((LANES,), jnp.float32)

    def _zrow(r, _):
        for k in range(D // LANES):
            rows_v[0, r, pl.ds(k * LANES, LANES)] = zero16
        return 0
    lax.fori_loop(0, B, _zrow, 0)

    row0 = pl.multiple_of(sid * ROWS_PT, 8)
    for t in range(ROWS_PT // B):
        pltpu.sync_copy(rows_v.at[0], acc_sh.at[pl.ds(row0 + t * B, B)])
    for t in range(5):
        pltpu.sync_copy(rows_v.at[0, 0],
                        den_sh.at[pl.ds(sid * 640 + t * D, D)])

    pltpu.sync_copy(c_hbm.at[pl.ds(pl.multiple_of(cid * 8 * D, 8), LANES)],
                    c_v)
    cvec = c_v[...]
    plsc.subcore_barrier()

    ebase = cid * E + sid * EPT
    cofs = cid * N

    def _stage_p1(j):
        b = lax.rem(j, 2)
        off = pl.multiple_of(ebase + j * B, 8)
        pltpu.sync_copy(u_hbm.at[pl.ds(off, B)], u_c)
        pltpu.sync_copy(v_hbm.at[pl.ds(off, B)], v2_c.at[b])

        def _idx(g, _2):
            o = pl.multiple_of(g * LANES, 8)
            ug2_c[b, pl.ds(o, LANES)] = u_c[pl.ds(o, LANES)] + cofs
            vg2_c[b, pl.ds(o, LANES)] = v2_c[b, pl.ds(o, LANES)] + cofs
            return 0
        lax.fori_loop(0, GPC, _idx, 0)
        pltpu.async_copy(el_hbm.at[ug2_c.at[b]], elu2_c.at[b], sem)
        pltpu.async_copy(er_hbm.at[vg2_c.at[b]], erv2_c.at[b], sem)

    _stage_p1(jnp.int32(0))

    def _p_chunk(j, _):
        b = lax.rem(j, 2)
        pltpu.make_async_copy(el_hbm.at[ug2_c.at[b]], elu2_c.at[b],
                              sem).wait()
        pltpu.make_async_copy(er_hbm.at[vg2_c.at[b]], erv2_c.at[b],
                              sem).wait()

        @pl.when(j < NCH - 1)
        def _():
            _stage_p1(j + 1)

        def _grp(g, _2):
            o = pl.multiple_of(g * LANES, 8)
            s = elu2_c[b, pl.ds(o, LANES)] + erv2_c[b, pl.ds(o, LANES)]
            s = jnp.where(s >= 0.0, s, 0.2 * s)
            p2_c[b, pl.ds(o, LANES)] = jnp.exp(s - cvec)
            return 0
        lax.fori_loop(0, GPC, _grp, 0)
        off = pl.multiple_of(ebase + j * B, 8)
        pltpu.sync_copy(p2_c.at[b], p_hbm.at[pl.ds(off, B)])
        pltpu.sync_copy(p2_c.at[b], den_sh.at[v2_c.at[b]], add=True)
        return 0
    lax.fori_loop(0, NCH, _p_chunk, 0)
    plsc.subcore_barrier()

    def _stage_p2(j):
        b = lax.rem(j, 2)
        off = pl.multiple_of(ebase + j * B, 8)
        pltpu.sync_copy(u_hbm.at[pl.ds(off, B)], u_c)
        pltpu.sync_copy(v_hbm.at[pl.ds(off, B)], v2_c.at[b])
        pltpu.sync_copy(p_hbm.at[pl.ds(off, B)], p2_c.at[b])

        def _idx(g, _2):
            o = pl.multiple_of(g * LANES, 8)
            ug2_c[b, pl.ds(o, LANES)] = u_c[pl.ds(o, LANES)] + cofs
            return 0
        lax.fori_loop(0, GPC, _idx, 0)
        pltpu.async_copy(feat_hbm.at[ug2_c.at[b]], rows_v.at[b], sem)

    _stage_p2(jnp.int32(0))

    def _agg_chunk(j, _):
        b = lax.rem(j, 2)
        pltpu.make_async_copy(feat_hbm.at[ug2_c.at[b]], rows_v.at[b],
                              sem).wait()

        @pl.when(j < NCH - 1)
        def _():
            _stage_p2(j + 1)

        def _scale(r, _2):
            rr = jnp.full((LANES,), r, jnp.int32)
            bb = jnp.full((LANES,), b, jnp.int32)
            af = plsc.load_gather(p2_c, [bb, rr])
            for k in range(D // LANES):
                sl = pl.ds(k * LANES, LANES)
                rows_v[b, r, sl] = rows_v[b, r, sl] * af
            return 0
        lax.fori_loop(0, B, _scale, 0)
        pltpu.sync_copy(rows_v.at[b], acc_sh.at[v2_c.at[b]], add=True)
        return 0
    lax.fori_loop(0, NCH, _agg_chunk, 0)
    plsc.subcore_barrier()

    pltpu.sync_copy(den_sh.at[pl.ds(sid * 640, 640)],
                    den_hbm.at[pl.ds(cid * DEN_PAD + sid * 640, 640)])

    @pl.when(sid < NS - 1)
    def _():
        hb = pl.multiple_of(cid * N + sid * ROWS_PT, 8)
        pltpu.sync_copy(acc_sh.at[pl.ds(row0, ROWS_PT)],
                        h_hbm.at[pl.ds(hb, ROWS_PT)])

    @pl.when(sid == NS - 1)
    def _():
        nrem = N - (NS - 1) * ROWS_PT
        hb = pl.multiple_of(cid * N + (NS - 1) * ROWS_PT, 8)
        pltpu.sync_copy(acc_sh.at[pl.ds(row0, nrem)],
                        h_hbm.at[pl.ds(hb, nrem)])


_sc_agg_built = None


def _sc_agg(*args):
    global _sc_agg_built
    if _sc_agg_built is None:
        _sc_agg_built = _build_sc_agg()
    return _sc_agg_built(*args)


def _build_sc_agg():
    return pl.kernel(
        _sc_body,
        out_type=(jax.ShapeDtypeStruct((2 * N, D), jnp.float32),
                  jax.ShapeDtypeStruct((2 * DEN_PAD,), jnp.float32),
                  jax.ShapeDtypeStruct((2 * E,), jnp.float32)),
        mesh=plsc.VectorSubcoreMesh(core_axis_name="c", subcore_axis_name="s",
                                    num_cores=2, num_subcores=NS),
        compiler_params=pltpu.CompilerParams(needs_layout_passes=False),
        scratch_types=[
            pltpu.VMEM((B,), jnp.int32),
            pltpu.VMEM((2, B), jnp.int32),
            pltpu.VMEM((2, B), jnp.int32),
            pltpu.VMEM((2, B), jnp.int32),
            pltpu.VMEM((2, B), jnp.float32),
            pltpu.VMEM((2, B), jnp.float32),
            pltpu.VMEM((2, B), jnp.float32),
            pltpu.VMEM((2, B, D), jnp.float32),
            pltpu.VMEM((LANES,), jnp.float32),
            pltpu.VMEM_SHARED((ACC_PAD, D), jnp.float32),
            pltpu.VMEM_SHARED((DEN_PAD,), jnp.float32),
            pltpu.SemaphoreType.DMA,
        ],
    )


def _tc_wl_body(x_ref, h0_ref, h1_ref, d0_ref, d1d_ref, wx_ref, wd_ref,
                b_ref, o_ref):
    cdims = (((1,), (1,)), ((), ()))
    den0 = d0_ref[...]
    den1 = d1d_ref[...]
    h0 = h0_ref[...] / jnp.where(den0 == 0.0, 1.0, den0)
    h1 = h1_ref[...] / jnp.where(den1 == 0.0, 1.0, den1)
    t = lax.dot_general(x_ref[...], wx_ref[...], cdims,
                        preferred_element_type=jnp.float32)
    bias = b_ref[...]
    a1 = h0 + bias
    d1 = jnp.maximum(t + lax.dot_general(a1, wd_ref[...], cdims,
                                         preferred_element_type=jnp.float32),
                     0.0)
    a2 = d1 + h1 + bias
    o_ref[...] = jnp.maximum(
        t + lax.dot_general(a2, wd_ref[...], cdims,
                            preferred_element_type=jnp.float32), 0.0)


_tc_wl = pl.pallas_call(
    _tc_wl_body,
    grid=(NBLK,),
    in_specs=[
        pl.BlockSpec((BN, D), lambda i: (i, 0)),
        pl.BlockSpec((BN, D), lambda i: (i, 0)),
        pl.BlockSpec((BN, D), lambda i: (NBLK + i, 0)),
        pl.BlockSpec((BN, 1), lambda i: (i, 0)),
        pl.BlockSpec((BN, 1), lambda i: (i, 0)),
        pl.BlockSpec((D, D), lambda i: (0, 0)),
        pl.BlockSpec((D, D), lambda i: (0, 0)),
        pl.BlockSpec((1, D), lambda i: (0, 0)),
    ],
    out_specs=pl.BlockSpec((BN, D), lambda i: (i, 0)),
    out_shape=jax.ShapeDtypeStruct((N, D), jnp.float32),
)


def kernel(x, edge_index0, edge_index1, W0, attn_l0, attn_r0,
           W1, attn_l1, attn_r1, wl_W, bias):
    Wst = jnp.stack([W0, W1])
    ALst = jnp.stack([attn_l0, attn_l1]).reshape(2, 1, D)
    ARst = jnp.stack([attn_r0, attn_r1]).reshape(2, 1, D)
    FEAT, EL, ER, CC = _tc_proj(x, Wst, ALst, ARst)
    U = jnp.concatenate([edge_index0[0], edge_index1[0]])
    V = jnp.concatenate([edge_index0[1], edge_index1[1]])
    H, DEN, _ = _sc_agg(U, V, EL.reshape(2 * N), ER.reshape(2 * N),
                        CC.reshape(16 * D), FEAT)
    den0 = DEN[:N].reshape(N, 1)
    den1 = DEN[DEN_PAD:DEN_PAD + N].reshape(N, 1)
    wlx = wl_W[:, :D]
    wld = wl_W[:, D:]
    return _tc_wl(x, H, H, den0, den1, wlx, wld, bias.reshape(1, D))


# speedup vs baseline: 26.5370x; 1.6681x over previous
import jax
import jax.numpy as jnp
from jax import lax
from jax.experimental import pallas as pl
from jax.experimental.pallas import tpu as pltpu
from jax.experimental.pallas import tpu_sc as plsc

N = 10000
E = 160000
D = 128
NS = 16
LANES = 16
EPT = E // NS
B = 80
NCH = EPT // B
GPC = B // LANES
ACC_PAD = 10240
ROWS_PT = ACC_PAD // NS
DEN_PAD = 10240
BN = 1000
NBLK = N // BN


def _tc_proj_body(x_ref, w_ref, al_ref, ar_ref,
                  feat_ref, el_ref, er_ref, c_ref, sm):
    i = pl.program_id(1)
    feat = lax.dot_general(x_ref[...], w_ref[0], (((1,), (1,)), ((), ())),
                           preferred_element_type=jnp.float32)
    feat_ref[...] = feat
    el = jnp.sum(feat * al_ref[0], axis=1, keepdims=True)
    er = jnp.sum(feat * ar_ref[0], axis=1, keepdims=True)
    el_ref[...] = el
    er_ref[...] = er
    bl = jnp.max(el)
    br = jnp.max(er)

    @pl.when(i == 0)
    def _():
        sm[0] = bl
        sm[1] = br

    @pl.when(i > 0)
    def _():
        sm[0] = jnp.maximum(sm[0], bl)
        sm[1] = jnp.maximum(sm[1], br)

    @pl.when(i == NBLK - 1)
    def _():
        c_ref[...] = jnp.full((8, D), jnp.maximum(sm[0] + sm[1], 0.0),
                              jnp.float32)


_tc_proj = pl.pallas_call(
    _tc_proj_body,
    grid=(2, NBLK),
    in_specs=[
        pl.BlockSpec((BN, D), lambda e, i: (i, 0)),
        pl.BlockSpec((1, D, D), lambda e, i: (e, 0, 0)),
        pl.BlockSpec((1, 1, D), lambda e, i: (e, 0, 0)),
        pl.BlockSpec((1, 1, D), lambda e, i: (e, 0, 0)),
    ],
    out_specs=[
        pl.BlockSpec((BN, D), lambda e, i: (e * NBLK + i, 0)),
        pl.BlockSpec((BN, 1), lambda e, i: (e * NBLK + i, 0)),
        pl.BlockSpec((BN, 1), lambda e, i: (e * NBLK + i, 0)),
        pl.BlockSpec((8, D), lambda e, i: (e, 0)),
    ],
    out_shape=[
        jax.ShapeDtypeStruct((2 * N, D), jnp.float32),
        jax.ShapeDtypeStruct((2 * N, 1), jnp.float32),
        jax.ShapeDtypeStruct((2 * N, 1), jnp.float32),
        jax.ShapeDtypeStruct((16, D), jnp.float32),
    ],
    scratch_shapes=[pltpu.SMEM((2,), jnp.float32)],
)


def _sc_body(u_hbm, v_hbm, el_hbm, er_hbm, c_hbm, feat_hbm,
             h_hbm, den_hbm,
             u_c, v2_c, ug2_c, vg2_c, p2_c, elu2_c, erv2_c, rows_v, c_v,
             acc_sh, den_sh, sem_e, sem_r, sem_d, sem_a):
    cid = lax.axis_index("c")
    sid = lax.axis_index("s")
    zero16 = jnp.zeros((LANES,), jnp.float32)

    def _zrow(r, _):
        for k in range(D // LANES):
            rows_v[0, r, pl.ds(k * LANES, LANES)] = zero16
        return 0
    lax.fori_loop(0, B, _zrow, 0)

    row0 = pl.multiple_of(sid * ROWS_PT, 8)
    for t in range(ROWS_PT // B):
        pltpu.sync_copy(rows_v.at[0], acc_sh.at[pl.ds(row0 + t * B, B)])
    for t in range(5):
        pltpu.sync_copy(rows_v.at[0, 0],
                        den_sh.at[pl.ds(sid * 640 + t * D, D)])

    pltpu.sync_copy(c_hbm.at[pl.ds(pl.multiple_of(cid * 8 * D, 8), LANES)],
                    c_v)
    cvec = c_v[...]
    plsc.subcore_barrier()

    ebase = cid * E + sid * EPT
    cofs = cid * N

    def _stage(j):
        b = lax.rem(j, 2)
        off = pl.multiple_of(ebase + j * B, 8)
        pltpu.sync_copy(u_hbm.at[pl.ds(off, B)], u_c)
        pltpu.sync_copy(v_hbm.at[pl.ds(off, B)], v2_c.at[b])

        def _idx(g, _2):
            o = pl.multiple_of(g * LANES, 8)
            ug2_c[b, pl.ds(o, LANES)] = u_c[pl.ds(o, LANES)] + cofs
            vg2_c[b, pl.ds(o, LANES)] = v2_c[b, pl.ds(o, LANES)] + cofs
            return 0
        lax.fori_loop(0, GPC, _idx, 0)
        pltpu.async_copy(el_hbm.at[ug2_c.at[b]], elu2_c.at[b], sem_e)
        pltpu.async_copy(er_hbm.at[vg2_c.at[b]], erv2_c.at[b], sem_e)
        pltpu.async_copy(feat_hbm.at[ug2_c.at[b]], rows_v.at[b], sem_r)

    _stage(jnp.int32(0))

    def _chunk(j, _):
        b = lax.rem(j, 2)
        b1 = 1 - b
        pltpu.make_async_copy(el_hbm.at[ug2_c.at[b]], elu2_c.at[b],
                              sem_e).wait()
        pltpu.make_async_copy(er_hbm.at[vg2_c.at[b]], erv2_c.at[b],
                              sem_e).wait()

        def _grp(g, _2):
            o = pl.multiple_of(g * LANES, 8)
            s = elu2_c[b, pl.ds(o, LANES)] + erv2_c[b, pl.ds(o, LANES)]
            s = jnp.where(s >= 0.0, s, 0.2 * s)
            p2_c[b, pl.ds(o, LANES)] = jnp.exp(s - cvec)
            return 0
        lax.fori_loop(0, GPC, _grp, 0)
        pltpu.make_async_copy(feat_hbm.at[ug2_c.at[b]], rows_v.at[b],
                              sem_r).wait()

        @pl.when(j < NCH - 1)
        def _():
            @pl.when(j > 0)
            def _():
                pltpu.make_async_copy(p2_c.at[b1],
                                      den_sh.at[v2_c.at[b1]], sem_d).wait()
                pltpu.make_async_copy(rows_v.at[b1],
                                      acc_sh.at[v2_c.at[b1]], sem_a).wait()
            _stage(j + 1)

        pltpu.async_copy(p2_c.at[b], den_sh.at[v2_c.at[b]], sem_d, add=True)

        def _scale(r, _2):
            rr = jnp.full((LANES,), r, jnp.int32)
            bb = jnp.full((LANES,), b, jnp.int32)
            af = plsc.load_gather(p2_c, [bb, rr])
            for k in range(D // LANES):
                sl = pl.ds(k * LANES, LANES)
                rows_v[b, r, sl] = rows_v[b, r, sl] * af
            return 0
        lax.fori_loop(0, B, _scale, 0)
        pltpu.async_copy(rows_v.at[b], acc_sh.at[v2_c.at[b]], sem_a,
                         add=True)
        return 0
    lax.fori_loop(0, NCH, _chunk, 0)

    for b in (0, 1):
        pltpu.make_async_copy(p2_c.at[b], den_sh.at[v2_c.at[b]],
                              sem_d).wait()
        pltpu.make_async_copy(rows_v.at[b], acc_sh.at[v2_c.at[b]],
                              sem_a).wait()
    plsc.subcore_barrier()

    pltpu.sync_copy(den_sh.at[pl.ds(sid * 640, 640)],
                    den_hbm.at[pl.ds(cid * DEN_PAD + sid * 640, 640)])

    @pl.when(sid < NS - 1)
    def _():
        hb = pl.multiple_of(cid * N + sid * ROWS_PT, 8)
        pltpu.sync_copy(acc_sh.at[pl.ds(row0, ROWS_PT)],
                        h_hbm.at[pl.ds(hb, ROWS_PT)])

    @pl.when(sid == NS - 1)
    def _():
        nrem = N - (NS - 1) * ROWS_PT
        hb = pl.multiple_of(cid * N + (NS - 1) * ROWS_PT, 8)
        pltpu.sync_copy(acc_sh.at[pl.ds(row0, nrem)],
                        h_hbm.at[pl.ds(hb, nrem)])


_sc_agg_built = None


def _sc_agg(*args):
    global _sc_agg_built
    if _sc_agg_built is None:
        _sc_agg_built = _build_sc_agg()
    return _sc_agg_built(*args)


def _build_sc_agg():
    return pl.kernel(
        _sc_body,
        out_type=(jax.ShapeDtypeStruct((2 * N, D), jnp.float32),
                  jax.ShapeDtypeStruct((2 * DEN_PAD,), jnp.float32)),
        mesh=plsc.VectorSubcoreMesh(core_axis_name="c", subcore_axis_name="s",
                                    num_cores=2, num_subcores=NS),
        compiler_params=pltpu.CompilerParams(needs_layout_passes=False),
        scratch_types=[
            pltpu.VMEM((B,), jnp.int32),
            pltpu.VMEM((2, B), jnp.int32),
            pltpu.VMEM((2, B), jnp.int32),
            pltpu.VMEM((2, B), jnp.int32),
            pltpu.VMEM((2, B), jnp.float32),
            pltpu.VMEM((2, B), jnp.float32),
            pltpu.VMEM((2, B), jnp.float32),
            pltpu.VMEM((2, B, D), jnp.float32),
            pltpu.VMEM((LANES,), jnp.float32),
            pltpu.VMEM_SHARED((ACC_PAD, D), jnp.float32),
            pltpu.VMEM_SHARED((DEN_PAD,), jnp.float32),
            pltpu.SemaphoreType.DMA,
            pltpu.SemaphoreType.DMA,
            pltpu.SemaphoreType.DMA,
            pltpu.SemaphoreType.DMA,
        ],
    )


def _tc_wl_body(x_ref, h0_ref, h1_ref, d0_ref, d1d_ref, wx_ref, wd_ref,
                b_ref, o_ref):
    cdims = (((1,), (1,)), ((), ()))
    den0 = d0_ref[...]
    den1 = d1d_ref[...]
    h0 = h0_ref[...] / jnp.where(den0 == 0.0, 1.0, den0)
    h1 = h1_ref[...] / jnp.where(den1 == 0.0, 1.0, den1)
    t = lax.dot_general(x_ref[...], wx_ref[...], cdims,
                        preferred_element_type=jnp.float32)
    bias = b_ref[...]
    a1 = h0 + bias
    d1 = jnp.maximum(t + lax.dot_general(a1, wd_ref[...], cdims,
                                         preferred_element_type=jnp.float32),
                     0.0)
    a2 = d1 + h1 + bias
    o_ref[...] = jnp.maximum(
        t + lax.dot_general(a2, wd_ref[...], cdims,
                            preferred_element_type=jnp.float32), 0.0)


_tc_wl = pl.pallas_call(
    _tc_wl_body,
    grid=(NBLK,),
    in_specs=[
        pl.BlockSpec((BN, D), lambda i: (i, 0)),
        pl.BlockSpec((BN, D), lambda i: (i, 0)),
        pl.BlockSpec((BN, D), lambda i: (NBLK + i, 0)),
        pl.BlockSpec((BN, 1), lambda i: (i, 0)),
        pl.BlockSpec((BN, 1), lambda i: (i, 0)),
        pl.BlockSpec((D, D), lambda i: (0, 0)),
        pl.BlockSpec((D, D), lambda i: (0, 0)),
        pl.BlockSpec((1, D), lambda i: (0, 0)),
    ],
    out_specs=pl.BlockSpec((BN, D), lambda i: (i, 0)),
    out_shape=jax.ShapeDtypeStruct((N, D), jnp.float32),
)


def kernel(x, edge_index0, edge_index1, W0, attn_l0, attn_r0,
           W1, attn_l1, attn_r1, wl_W, bias):
    Wst = jnp.stack([W0, W1])
    ALst = jnp.stack([attn_l0, attn_l1]).reshape(2, 1, D)
    ARst = jnp.stack([attn_r0, attn_r1]).reshape(2, 1, D)
    FEAT, EL, ER, CC = _tc_proj(x, Wst, ALst, ARst)
    U = jnp.concatenate([edge_index0[0], edge_index1[0]])
    V = jnp.concatenate([edge_index0[1], edge_index1[1]])
    H, DEN = _sc_agg(U, V, EL.reshape(2 * N), ER.reshape(2 * N),
                     CC.reshape(16 * D), FEAT)
    den0 = DEN[:N].reshape(N, 1)
    den1 = DEN[DEN_PAD:DEN_PAD + N].reshape(N, 1)
    wlx = wl_W[:, :D]
    wld = wl_W[:, D:]
    return _tc_wl(x, H, H, den0, den1, wlx, wld, bias.reshape(1, D))
